# baseline (device time: 356084 ns/iter reference)
import jax
import jax.numpy as jnp
from jax import lax
from jax.experimental import pallas as pl
from jax.experimental.pallas import tpu as pltpu

N_DEV = 8
RS_HOPS = N_DEV - 1
N_HOPS = 2 * (N_DEV - 1)
N_SUB = 4


def _perm(p):
    return jnp.where(p < 4, p, 11 - p)


def kernel(x):
    m, n = x.shape
    chunk = m // N_DEV
    half = chunk // 2
    quart = half // N_SUB

    def body(x_ref, out_ref, comm_r, comm_l,
             send_r, recv_r, send_l, recv_l,
             credit_r, credit_l, copy_r, copy_l):
        my = lax.axis_index("i")
        r = _perm(my)
        right = _perm(jnp.mod(r + 1, N_DEV))
        left = _perm(jnp.mod(r - 1, N_DEV))

        barrier_sem = pltpu.get_barrier_semaphore()
        for nbr in (left, right):
            pl.semaphore_signal(
                barrier_sem, inc=1,
                device_id=(nbr,), device_id_type=pl.DeviceIdType.MESH,
            )
        pl.semaphore_wait(barrier_sem, 2)

        def top(ref, c, q):
            return ref.at[pl.ds(c * chunk + q * quart, quart), :]

        def bot(ref, c, q):
            return ref.at[pl.ds(c * chunk + half + q * quart, quart), :]

        streams = []
        for q in range(N_SUB):
            streams.append(dict(
                comm=comm_r, send=send_r, recv=recv_r, credit=credit_r,
                copy=copy_r, slc=top, dst=right, ups=left, sgn=+1, q=q,
            ))
            streams.append(dict(
                comm=comm_l, send=send_l, recv=recv_l, credit=credit_l,
                copy=copy_l, slc=bot, dst=left, ups=right, sgn=-1, q=q,
            ))

        def make_rdma(s, h):
            slot = h % 2
            q = s["q"]
            if h == 0:
                src = s["slc"](x_ref, r, q)
            else:
                src = s["comm"].at[(h - 1) % 2, q]
            return pltpu.make_async_remote_copy(
                src_ref=src,
                dst_ref=s["comm"].at[slot, q],
                send_sem=s["send"].at[slot, q],
                recv_sem=s["recv"].at[slot, q],
                device_id=(s["dst"],),
                device_id_type=pl.DeviceIdType.MESH,
            )

        def start_hop(s, h):
            if h >= 2:
                pl.semaphore_wait(s["credit"].at[s["q"]], 1)
            rdma = make_rdma(s, h)
            rdma.start()
            s["rdma"] = rdma

        for s in streams:
            start_hop(s, 0)

        for h in range(N_HOPS):
            slot = h % 2
            for s in streams:
                q = s["q"]
                rdma = s["rdma"]
                rdma.wait_send()
                cp = s.pop("cp", None)
                if cp is not None:
                    cp.wait()
                if 1 <= h < N_HOPS - 1:
                    pl.semaphore_signal(
                        s["credit"].at[q], inc=1,
                        device_id=(s["ups"],),
                        device_id_type=pl.DeviceIdType.MESH,
                    )
                rdma.wait_recv()
                if h < RS_HOPS:
                    c = jnp.mod(r - s["sgn"] * (h + 1), N_DEV)
                    s["comm"][slot, q] = (
                        s["comm"][slot, q] + s["slc"](x_ref, c, q)[...]
                    )
                    if h == RS_HOPS - 1:
                        own = jnp.mod(r + s["sgn"], N_DEV)
                        cp = pltpu.make_async_copy(
                            s["comm"].at[slot, q],
                            s["slc"](out_ref, own, q),
                            s["copy"].at[q],
                        )
                        cp.start()
                        s["cp"] = cp
                else:
                    k = h - RS_HOPS
                    c = jnp.mod(r - s["sgn"] * k, N_DEV)
                    cp = pltpu.make_async_copy(
                        s["comm"].at[slot, q],
                        s["slc"](out_ref, c, q),
                        s["copy"].at[q],
                    )
                    cp.start()
                    s["cp"] = cp
                if h + 1 < N_HOPS:
                    start_hop(s, h + 1)

        for s in streams:
            cp = s.pop("cp", None)
            if cp is not None:
                cp.wait()

    return pl.pallas_call(
        body,
        out_shape=jax.ShapeDtypeStruct((m, n), x.dtype),
        in_specs=[pl.BlockSpec(memory_space=pltpu.VMEM)],
        out_specs=pl.BlockSpec(memory_space=pltpu.MemorySpace.HBM),
        scratch_shapes=[
            pltpu.VMEM((2, N_SUB, quart, n), x.dtype),
            pltpu.VMEM((2, N_SUB, quart, n), x.dtype),
            pltpu.SemaphoreType.DMA((2, N_SUB)),
            pltpu.SemaphoreType.DMA((2, N_SUB)),
            pltpu.SemaphoreType.DMA((2, N_SUB)),
            pltpu.SemaphoreType.DMA((2, N_SUB)),
            pltpu.SemaphoreType.REGULAR((N_SUB,)),
            pltpu.SemaphoreType.REGULAR((N_SUB,)),
            pltpu.SemaphoreType.DMA((N_SUB,)),
            pltpu.SemaphoreType.DMA((N_SUB,)),
        ],
        compiler_params=pltpu.CompilerParams(
            collective_id=0,
            vmem_limit_bytes=48 * 1024 * 1024,
        ),
    )(x)


# device time: 346047 ns/iter; 1.0290x vs baseline; 1.0290x over previous
import jax
import jax.numpy as jnp
from jax import lax
from jax.experimental import pallas as pl
from jax.experimental.pallas import tpu as pltpu

N_DEV = 8
RS_HOPS = N_DEV - 1
N_HOPS = 2 * (N_DEV - 1)
N_SUB = 4


def _perm(p):
    return jnp.where(p < 4, p, 11 - p)


def kernel(x):
    m, n = x.shape
    chunk = m // N_DEV
    half = chunk // 2
    quart = half // N_SUB

    def body(x_ref, out_ref, comm_r, comm_l,
             send_r, recv_r, send_l, recv_l,
             credit_r, credit_l, copy_r, copy_l,
             xbuf, xsem):
        my = lax.axis_index("i")
        r = _perm(my)
        right = _perm(jnp.mod(r + 1, N_DEV))
        left = _perm(jnp.mod(r - 1, N_DEV))

        barrier_sem = pltpu.get_barrier_semaphore()
        for nbr in (left, right):
            pl.semaphore_signal(
                barrier_sem, inc=1,
                device_id=(nbr,), device_id_type=pl.DeviceIdType.MESH,
            )
        pl.semaphore_wait(barrier_sem, 2)

        def top(ref, c, q):
            return ref.at[pl.ds(c * chunk + q * quart, quart), :]

        def bot(ref, c, q):
            return ref.at[pl.ds(c * chunk + half + q * quart, quart), :]

        streams = []
        for q in range(N_SUB):
            streams.append(dict(
                comm=comm_r, send=send_r, recv=recv_r, credit=credit_r,
                copy=copy_r, slc=top, dst=right, ups=left, sgn=+1, q=q,
            ))
            streams.append(dict(
                comm=comm_l, send=send_l, recv=recv_l, credit=credit_l,
                copy=copy_l, slc=bot, dst=left, ups=right, sgn=-1, q=q,
            ))
        for si, s in enumerate(streams):
            s["si"] = si

        def xfetch(s, h):
            c = jnp.mod(r - s["sgn"] * (h + 1), N_DEV)
            return pltpu.make_async_copy(
                s["slc"](x_ref, c, s["q"]),
                xbuf.at[s["si"], h % 2],
                xsem.at[s["si"], h % 2],
            )

        def make_rdma(s, h):
            slot = h % 2
            q = s["q"]
            if h == 0:
                src = s["slc"](x_ref, r, q)
            else:
                src = s["comm"].at[(h - 1) % 2, q]
            return pltpu.make_async_remote_copy(
                src_ref=src,
                dst_ref=s["comm"].at[slot, q],
                send_sem=s["send"].at[slot, q],
                recv_sem=s["recv"].at[slot, q],
                device_id=(s["dst"],),
                device_id_type=pl.DeviceIdType.MESH,
            )

        def start_hop(s, h):
            if h >= 2:
                pl.semaphore_wait(s["credit"].at[s["q"]], 1)
            rdma = make_rdma(s, h)
            rdma.start()
            s["rdma"] = rdma

        for s in streams:
            start_hop(s, 0)
            xfetch(s, 0).start()

        for h in range(N_HOPS):
            slot = h % 2
            for s in streams:
                q = s["q"]
                rdma = s["rdma"]
                rdma.wait_send()
                cp = s.pop("cp", None)
                if cp is not None:
                    cp.wait()
                if 1 <= h < N_HOPS - 1:
                    pl.semaphore_signal(
                        s["credit"].at[q], inc=1,
                        device_id=(s["ups"],),
                        device_id_type=pl.DeviceIdType.MESH,
                    )
                if h + 1 < RS_HOPS:
                    xfetch(s, h + 1).start()
                rdma.wait_recv()
                if h < RS_HOPS:
                    xfetch(s, h).wait()
                    s["comm"][slot, q] = (
                        s["comm"][slot, q] + xbuf[s["si"], slot]
                    )
                    if h == RS_HOPS - 1:
                        own = jnp.mod(r + s["sgn"], N_DEV)
                        cp = pltpu.make_async_copy(
                            s["comm"].at[slot, q],
                            s["slc"](out_ref, own, q),
                            s["copy"].at[q],
                        )
                        cp.start()
                        s["cp"] = cp
                else:
                    k = h - RS_HOPS
                    c = jnp.mod(r - s["sgn"] * k, N_DEV)
                    cp = pltpu.make_async_copy(
                        s["comm"].at[slot, q],
                        s["slc"](out_ref, c, q),
                        s["copy"].at[q],
                    )
                    cp.start()
                    s["cp"] = cp
                if h + 1 < N_HOPS:
                    start_hop(s, h + 1)

        for s in streams:
            cp = s.pop("cp", None)
            if cp is not None:
                cp.wait()

    return pl.pallas_call(
        body,
        out_shape=jax.ShapeDtypeStruct((m, n), x.dtype),
        in_specs=[pl.BlockSpec(memory_space=pl.ANY)],
        out_specs=pl.BlockSpec(memory_space=pltpu.MemorySpace.HBM),
        scratch_shapes=[
            pltpu.VMEM((2, N_SUB, quart, n), x.dtype),
            pltpu.VMEM((2, N_SUB, quart, n), x.dtype),
            pltpu.SemaphoreType.DMA((2, N_SUB)),
            pltpu.SemaphoreType.DMA((2, N_SUB)),
            pltpu.SemaphoreType.DMA((2, N_SUB)),
            pltpu.SemaphoreType.DMA((2, N_SUB)),
            pltpu.SemaphoreType.REGULAR((N_SUB,)),
            pltpu.SemaphoreType.REGULAR((N_SUB,)),
            pltpu.SemaphoreType.DMA((N_SUB,)),
            pltpu.SemaphoreType.DMA((N_SUB,)),
            pltpu.VMEM((2 * N_SUB, 2, quart, n), x.dtype),
            pltpu.SemaphoreType.DMA((2 * N_SUB, 2)),
        ],
        compiler_params=pltpu.CompilerParams(
            collective_id=0,
            vmem_limit_bytes=48 * 1024 * 1024,
        ),
    )(x)
